# baseline (speedup 1.0000x reference)
import functools

import jax
import jax.numpy as jnp
from jax import lax
from jax.experimental import pallas as pl
from jax.experimental.pallas import tpu as pltpu
from jax.experimental.pallas import tpu_sc as plsc

_N_H = 8
_N = 64
_NUM_CLS = 117
_NUM_OBJ = 80
_HUMAN = 49
_DIM = 300
_PAIRS = _N_H * _N - _N_H
_NW = 32


def _l2n(x):
    return x / jnp.maximum(jnp.sqrt(jnp.sum(x * x, axis=-1, keepdims=True)),
                           1e-12)


def _prep_body(lab_ref, ent_ref, rel_ref, nv_ref, oh_ref,
               hp_o, rn_o, wn_o, tp_o, s_o):
    lab = jnp.where(lax.broadcasted_iota(jnp.int32, (_N, 1), 0) < _N_H,
                    _HUMAN, lab_ref[...])
    oh64 = (lab == lax.broadcasted_iota(jnp.int32, (_N, _NUM_OBJ), 1)
            ).astype(jnp.float32)
    ent = ent_ref[...]
    tn = _l2n(jnp.dot(oh64, ent, preferred_element_type=jnp.float32))
    hn = _l2n(ent[_HUMAN:_HUMAN + 1, :])
    wn = _l2n(nv_ref[...])
    rn = _l2n(rel_ref[...])
    hp = hn - jnp.sum(hn * wn, axis=-1, keepdims=True) * wn
    hp_o[...] = hp
    rn_o[...] = rn
    wn_o[...] = wn
    d = lax.dot_general(tn, wn, (((1,), (1,)), ((), ())),
                        preferred_element_type=jnp.float32)
    tp = tn[:, None, :] - d[:, :, None] * wn[None, :, :]
    tp_o[...] = tp
    diff = (hp + rn)[None, :, :] - tp
    s = jnp.sqrt(jnp.sum(diff * diff, axis=-1))
    s_o[...] = jnp.dot(oh_ref[...], s, preferred_element_type=jnp.float32)


def _prep(box_labels, ent_emb, rel_emb, norm_vec, oh504):
    small = jax.ShapeDtypeStruct((_NUM_CLS, _DIM), jnp.float32)
    return pl.pallas_call(
        _prep_body,
        out_shape=(small, small, small,
                   jax.ShapeDtypeStruct((_N, _NUM_CLS, _DIM), jnp.float32),
                   jax.ShapeDtypeStruct((_PAIRS, _NUM_CLS), jnp.float32)),
    )(box_labels.reshape(_N, 1), ent_emb, rel_emb, norm_vec, oh504)


_BIG = jax.ShapeDtypeStruct((_PAIRS, _NUM_CLS, _DIM), jnp.float32)


@functools.partial(
    pl.kernel,
    out_type=[_BIG, _BIG, _BIG, _BIG],
    mesh=plsc.VectorSubcoreMesh(core_axis_name="c", subcore_axis_name="s"),
    scratch_types=[
        pltpu.VMEM_SHARED((3, _NUM_CLS, _DIM), jnp.float32),
    ],
)
def _expand(hp_hbm, rn_hbm, wn_hbm, tp_hbm,
            h_out, r_out, w_out, t_out, tab_s):
    cid = lax.axis_index("c")
    sid = lax.axis_index("s")
    wid = sid * 2 + cid
    start = wid * 63 // 4
    end = (wid + 1) * 63 // 4

    @pl.when(sid == 0)
    def _load():
        pltpu.sync_copy(hp_hbm, tab_s.at[0])
        pltpu.sync_copy(rn_hbm, tab_s.at[1])
        pltpu.sync_copy(wn_hbm, tab_s.at[2])

    plsc.subcore_barrier()

    def body(r, carry):
        x = r // (_N - 1)
        j = r - x * (_N - 1)
        y = j + jnp.where(j >= x, 1, 0).astype(jnp.int32)
        pltpu.sync_copy(tab_s.at[0], h_out.at[r])
        pltpu.sync_copy(tab_s.at[1], r_out.at[r])
        pltpu.sync_copy(tab_s.at[2], w_out.at[r])
        pltpu.sync_copy(tp_hbm.at[y], t_out.at[r])
        return carry

    lax.fori_loop(start, end, body, 0)


def _static_onehot():
    import numpy as np
    ys = np.array([j + (1 if j >= x else 0)
                   for x in range(_N_H) for j in range(_N - 1)], np.int32)
    return (ys[:, None] == np.arange(_N)[None, :]).astype(np.float32)


_OH504 = _static_onehot()


def kernel(box_labels, ent_emb, rel_emb, norm_vec):
    hp, rn, wn, tp, scores = _prep(box_labels, ent_emb, rel_emb, norm_vec,
                                   jnp.asarray(_OH504))
    h_keep, r_keep, w_keep, t_keep = _expand(hp, rn, wn, tp)
    return (h_keep, r_keep, w_keep, t_keep, scores)
